# baseline (speedup 1.0000x reference)
import functools

import jax
import jax.numpy as jnp
from jax import lax
from jax.experimental import pallas as pl
from jax.experimental.pallas import tpu as pltpu
from jax.experimental.pallas import tpu_sc as plsc

_BR = 32
_BC = 2048


def _sc_body(labt_hbm, out_hbm, lab_v, blk_v, *, b, l):
    wid = lax.axis_index("s") * 2 + lax.axis_index("c")
    big = wid < 29
    g0 = jnp.where(big, 4 * wid, 116 + 3 * (wid - 29))
    ng = jnp.where(big, 4, 3)
    r0 = 8 * g0
    nhalf = b // _BC
    iters = _BC // 16

    def _chunk(t, _):
        j = t // nhalf
        h = t % nhalf
        pltpu.sync_copy(labt_hbm.at[j, 0, pl.ds(h * _BC, _BC)], lab_v)

        def _vec(i, _):
            lab = lab_v[pl.ds(i * 16, 16)]
            one = jnp.float32(1)
            zero = jnp.float32(0)
            for r in range(24):
                blk_v[r, pl.ds(i * 16, 16)] = jnp.where(
                    lab == r0 + r, one, zero
                )

            @pl.when(ng == 4)
            def _high_rows():
                for r in range(24, _BR):
                    blk_v[r, pl.ds(i * 16, 16)] = jnp.where(
                        lab == r0 + r, one, zero
                    )

            return 0

        lax.fori_loop(0, iters, _vec, 0)

        @pl.when(ng == 4)
        def _copy_big():
            pltpu.sync_copy(
                blk_v, out_hbm.at[j, pl.ds(r0, _BR), pl.ds(h * _BC, _BC)]
            )

        @pl.when(ng == 3)
        def _copy_small():
            pltpu.sync_copy(
                blk_v.at[pl.ds(0, 24)],
                out_hbm.at[j, pl.ds(r0, 24), pl.ds(h * _BC, _BC)],
            )

        return 0

    lax.fori_loop(0, l * nhalf, _chunk, 0)


def kernel(label, N):
    n_cls = 1000
    b, l = label.shape
    labt3 = label.T.reshape(l, 1, b)
    mesh = plsc.VectorSubcoreMesh(core_axis_name="c", subcore_axis_name="s")
    k = pl.kernel(
        functools.partial(_sc_body, b=b, l=l),
        out_type=jax.ShapeDtypeStruct((l, n_cls, b), jnp.float32),
        mesh=mesh,
        scratch_types=[
            pltpu.VMEM((_BC,), jnp.int32),
            pltpu.VMEM((_BR, _BC), jnp.float32),
        ],
    )
    out = k(labt3)
    return out.transpose(1, 2, 0)


# speedup vs baseline: 2.9080x; 2.9080x over previous
import functools

import jax
import jax.numpy as jnp
from jax.experimental import pallas as pl


def _body(lab_ref, out_ref, *, n_cls):
    j = pl.program_id(0)
    row = lab_ref[pl.ds(j, 1), :]
    cls = jax.lax.broadcasted_iota(jnp.int32, (n_cls, 1), 0)
    out_ref[0] = (row == cls).astype(jnp.float32)


def kernel(label, N):
    n_cls = 1000
    b, l = label.shape
    lab_t = label.T
    out = pl.pallas_call(
        functools.partial(_body, n_cls=n_cls),
        grid=(l,),
        in_specs=[pl.BlockSpec((l, b), lambda j: (0, 0))],
        out_specs=pl.BlockSpec((1, n_cls, b), lambda j: (j, 0, 0)),
        out_shape=jax.ShapeDtypeStruct((l, n_cls, b), jnp.float32),
    )(lab_t)
    return out.transpose(1, 2, 0)


# speedup vs baseline: 2.9963x; 1.0304x over previous
import functools

import jax
import jax.numpy as jnp
from jax.experimental import pallas as pl

_BLKC = 200


def _body(lab_ref, out_ref):
    j = pl.program_id(0)
    cb = pl.program_id(1)
    row = lab_ref[pl.ds(j, 1), :]
    cls = jax.lax.broadcasted_iota(jnp.int32, (_BLKC, 1), 0) + cb * _BLKC
    out_ref[0] = (row == cls).astype(jnp.float32)


def kernel(label, N):
    n_cls = 1000
    b, l = label.shape
    lab_t = label.T
    out = pl.pallas_call(
        _body,
        grid=(l, n_cls // _BLKC),
        in_specs=[pl.BlockSpec((l, b), lambda j, cb: (0, 0))],
        out_specs=pl.BlockSpec((1, _BLKC, b), lambda j, cb: (j, cb, 0)),
        out_shape=jax.ShapeDtypeStruct((l, n_cls, b), jnp.float32),
    )(lab_t)
    return out.transpose(1, 2, 0)
